# baseline (speedup 1.0000x reference)
import functools

import jax
import jax.numpy as jnp
from jax import lax
from jax.experimental import pallas as pl
from jax.experimental.pallas import tpu as pltpu
from jax.experimental.pallas import tpu_sc as plsc

_CH = 2048


def _dense_body(media_ref, WbigT_ref, bbig_ref, WwT_ref, bw_ref,
                W2T_ref, b2_ref, W3T_ref, b3_ref, et_ref, *, h):
    x = media_ref[...]
    ag = jnp.dot(x, WbigT_ref[...], preferred_element_type=jnp.float32) \
        + bbig_ref[...]
    a = jnp.tanh(ag[:, :h])
    g = jax.nn.sigmoid(ag[:, h:2 * h])
    h1 = jax.nn.gelu(ag[:, 2 * h:])
    s = jnp.dot(a * g, WwT_ref[...], preferred_element_type=jnp.float32) \
        + bw_ref[...]
    h2 = jax.nn.gelu(jnp.dot(h1, W2T_ref[...],
                             preferred_element_type=jnp.float32) + b2_ref[...])
    logit = jnp.dot(h2, W3T_ref[...], preferred_element_type=jnp.float32) \
        + b3_ref[...]
    e = jnp.exp(s)
    et_ref[...] = jnp.concatenate([e, e * logit], axis=1)


def _sc_agg_body(et_hbm, cu_hbm, out_hbm, cu_v, buf, acc_e_v, acc_el_v,
                 *, n_tok, nseg):
    wid = lax.axis_index("s") * 2 + lax.axis_index("c")
    cls = wid % 2
    j = wid // 2
    pltpu.sync_copy(cu_hbm, cu_v)
    jv = jnp.full((16,), j, jnp.int32)
    lov = plsc.load_gather(cu_v, [jv])
    hiv = plsc.load_gather(cu_v, [jv + 1])
    lo = jnp.max(lov)
    hi = jnp.max(hiv)
    loA = lo & ~7
    loAv = lov & ~7
    nch = (hi - loA + _CH - 1) // _CH
    iota = lax.iota(jnp.int32, 16)
    colc = jnp.full((16,), cls, jnp.int32)

    def chunk_body(ic, carry):
        acc_e, acc_el = carry
        nom = loA + ic * _CH
        start = pl.multiple_of(jnp.minimum(nom, n_tok - _CH), 8)
        pltpu.sync_copy(et_hbm.at[pl.ds(start * 4, _CH * 4)], buf)
        nomv = loAv + ic * _CH
        for kv in range(_CH // 16):
            tokv = start + 16 * kv + iota
            flat = (16 * kv + iota) * 4 + colc
            ev = plsc.load_gather(buf, [flat])
            elv = plsc.load_gather(buf, [flat + 2])
            m = (tokv >= lov) & (tokv < hiv) \
                & (tokv >= nomv) & (tokv < nomv + _CH)
            acc_e = acc_e + jnp.where(m, ev, 0.0)
            acc_el = acc_el + jnp.where(m, elv, 0.0)
        return acc_e, acc_el

    z16 = jnp.zeros((16,), jnp.float32)
    acc_e, acc_el = lax.fori_loop(0, nch, chunk_body, (z16, z16))
    acc_e_v[...] = acc_e
    acc_el_v[...] = acc_el
    off_e = pl.multiple_of((cls * nseg + j) * 16, 16)
    off_el = pl.multiple_of(((cls + 2) * nseg + j) * 16, 16)
    pltpu.sync_copy(acc_e_v, out_hbm.at[pl.ds(off_e, 16)])
    pltpu.sync_copy(acc_el_v, out_hbm.at[pl.ds(off_el, 16)])


def kernel(media, cu_seqlens, Wa, ba, Wg, bg, Ww, bw, W1, b1, W2, b2, W3, b3,
           output_scale, output_bias):
    n_tok, d = media.shape
    nseg = cu_seqlens.shape[0] - 1
    ncls = Ww.shape[0]
    h = Wa.shape[0]
    d1 = W1.shape[0]
    d2 = W2.shape[0]
    blk = 1024
    nsteps = n_tok // blk
    dbig = 2 * h + d1

    row = lambda v: v.reshape(1, -1)
    WbigT = jnp.concatenate([Wa.T, Wg.T, W1.T], axis=1)
    bbig = jnp.concatenate([ba, bg, b1])
    const = lambda shape: pl.BlockSpec(shape, lambda i: (0, 0))
    et = pl.pallas_call(
        functools.partial(_dense_body, h=h),
        grid=(nsteps,),
        in_specs=[
            pl.BlockSpec((blk, d), lambda i: (i, 0)),
            const((d, dbig)), const((1, dbig)),
            const((d, ncls)), const((1, ncls)),
            const((d1, d2)), const((1, d2)),
            const((d2, ncls)), const((1, ncls)),
        ],
        out_specs=pl.BlockSpec((blk, 2 * ncls), lambda i: (i, 0)),
        out_shape=jax.ShapeDtypeStruct((n_tok, 2 * ncls), jnp.float32),
    )(media, WbigT, row(bbig), Ww.T, row(bw), W2.T, row(b2), W3.T, row(b3))

    cu_pad = jnp.pad(cu_seqlens, (0, 128 - cu_seqlens.shape[0]))
    sc_agg = pl.kernel(
        functools.partial(_sc_agg_body, n_tok=n_tok, nseg=nseg),
        out_type=jax.ShapeDtypeStruct((2 * ncls * nseg * 16,), jnp.float32),
        mesh=plsc.VectorSubcoreMesh(core_axis_name="c", subcore_axis_name="s"),
        compiler_params=pltpu.CompilerParams(needs_layout_passes=False),
        scratch_types=[
            pltpu.VMEM((128,), jnp.int32),
            pltpu.VMEM((_CH * 2 * ncls,), jnp.float32),
            pltpu.VMEM((16,), jnp.float32),
            pltpu.VMEM((16,), jnp.float32),
        ],
    )
    p = sc_agg(et.reshape(-1), cu_pad).reshape(2 * ncls, nseg, 16)
    z = p[:ncls].sum(-1)
    o = p[ncls:].sum(-1)
    out = jnp.where(z > 0, o / z, 0.0)
    return out.T * output_scale + output_bias


# speedup vs baseline: 1.2561x; 1.2561x over previous
import functools

import jax
import jax.numpy as jnp
from jax import lax
from jax.experimental import pallas as pl
from jax.experimental.pallas import tpu as pltpu


def _fused_body(media_ref, WbigT_ref, bbig_ref, WwT_ref, bw_ref,
                W2T_ref, b2_ref, W3T_ref, b3_ref, start_ref, end_ref,
                out_ref, zo_ref, *, blk, nsteps, nseg, ncls, h, d1):
    i = pl.program_id(0)

    @pl.when(i == 0)
    def _init():
        zo_ref[...] = jnp.zeros((2 * ncls, nseg), jnp.float32)

    x = media_ref[...]
    ag = jnp.dot(x, WbigT_ref[...], preferred_element_type=jnp.float32) \
        + bbig_ref[...]
    a = jnp.tanh(ag[:, :h])
    g = jax.nn.sigmoid(ag[:, h:2 * h])
    h1 = jax.nn.gelu(ag[:, 2 * h:])
    s = jnp.dot(a * g, WwT_ref[...], preferred_element_type=jnp.float32) \
        + bw_ref[...]
    h2 = jax.nn.gelu(jnp.dot(h1, W2T_ref[...],
                             preferred_element_type=jnp.float32) + b2_ref[...])
    logit = jnp.dot(h2, W3T_ref[...], preferred_element_type=jnp.float32) \
        + b3_ref[...]

    tok = i * blk + lax.broadcasted_iota(jnp.int32, (blk, nseg), 0)
    onehot = ((tok >= start_ref[...]) & (tok < end_ref[...])) \
        .astype(jnp.float32)

    e = jnp.exp(s)
    q = jnp.concatenate([e, e * logit], axis=1)
    zo_ref[...] += lax.dot_general(q, onehot, (((0,), (0,)), ((), ())),
                                   preferred_element_type=jnp.float32)

    @pl.when(i == nsteps - 1)
    def _fin():
        z = zo_ref[:ncls, :]
        o = zo_ref[ncls:, :]
        out_ref[...] = jnp.where(z > 0, o / z, 0.0)


def kernel(media, cu_seqlens, Wa, ba, Wg, bg, Ww, bw, W1, b1, W2, b2, W3, b3,
           output_scale, output_bias):
    n_tok, d = media.shape
    nseg = cu_seqlens.shape[0] - 1
    ncls = Ww.shape[0]
    h = Wa.shape[0]
    d1 = W1.shape[0]
    d2 = W2.shape[0]
    blk = 1024
    nsteps = n_tok // blk
    dbig = 2 * h + d1

    body = functools.partial(_fused_body, blk=blk, nsteps=nsteps, nseg=nseg,
                             ncls=ncls, h=h, d1=d1)
    row = lambda v: v.reshape(1, -1)
    WbigT = jnp.concatenate([Wa.T, Wg.T, W1.T], axis=1)
    bbig = jnp.concatenate([ba, bg, b1])
    start = cu_seqlens[:nseg].reshape(1, nseg)
    end = cu_seqlens[1:].reshape(1, nseg)
    const = lambda shape: pl.BlockSpec(shape, lambda i: (0, 0))
    out = pl.pallas_call(
        body,
        grid=(nsteps,),
        in_specs=[
            pl.BlockSpec((blk, d), lambda i: (i, 0)),
            const((d, dbig)), const((1, dbig)),
            const((d, ncls)), const((1, ncls)),
            const((d1, d2)), const((1, d2)),
            const((d2, ncls)), const((1, ncls)),
            const((1, nseg)), const((1, nseg)),
        ],
        out_specs=pl.BlockSpec((ncls, nseg), lambda i: (0, 0)),
        out_shape=jax.ShapeDtypeStruct((ncls, nseg), jnp.float32),
        scratch_shapes=[pltpu.VMEM((2 * ncls, nseg), jnp.float32)],
    )(media, WbigT, row(bbig), Ww.T, row(bw),
      W2.T, row(b2), W3.T, row(b3), start, end)
    return out.T * output_scale + output_bias


# speedup vs baseline: 1.3017x; 1.0363x over previous
import functools

import jax
import jax.numpy as jnp
from jax import lax
from jax.experimental import pallas as pl
from jax.experimental.pallas import tpu as pltpu


def _fused_body(media_ref, WbigT_ref, bbig_ref, WwT_ref, bw_ref,
                W2T_ref, b2_ref, W3T_ref, b3_ref, start_ref, end_ref,
                out_ref, zo_ref, *, blk, nsteps, nseg, ncls, h, d1):
    i = pl.program_id(0)

    @pl.when(i == 0)
    def _init():
        zo_ref[...] = jnp.zeros((2 * ncls, nseg), jnp.float32)

    x = media_ref[...]
    ag = jnp.dot(x, WbigT_ref[...], preferred_element_type=jnp.float32) \
        + bbig_ref[...]
    a = jnp.tanh(ag[:, :h])
    g = 0.5 * (1.0 + jnp.tanh(ag[:, h:2 * h] * 0.5))
    h1 = jax.nn.gelu(ag[:, 2 * h:])
    s = jnp.dot(a * g, WwT_ref[...], preferred_element_type=jnp.float32) \
        + bw_ref[...]
    h2 = jax.nn.gelu(jnp.dot(h1, W2T_ref[...],
                             preferred_element_type=jnp.float32) + b2_ref[...])
    logit = jnp.dot(h2, W3T_ref[...], preferred_element_type=jnp.float32) \
        + b3_ref[...]

    tok = i * blk + lax.broadcasted_iota(jnp.int32, (blk, nseg), 0)
    onehot = ((tok >= start_ref[...]) & (tok < end_ref[...])) \
        .astype(jnp.float32)

    e = jnp.exp(s)
    q = jnp.concatenate([e, e * logit], axis=1)
    zo_ref[...] += lax.dot_general(q, onehot, (((0,), (0,)), ((), ())),
                                   preferred_element_type=jnp.float32)

    @pl.when(i == nsteps - 1)
    def _fin():
        z = zo_ref[:ncls, :]
        o = zo_ref[ncls:, :]
        out_ref[...] = jnp.where(z > 0, o / z, 0.0)


def kernel(media, cu_seqlens, Wa, ba, Wg, bg, Ww, bw, W1, b1, W2, b2, W3, b3,
           output_scale, output_bias):
    n_tok, d = media.shape
    nseg = cu_seqlens.shape[0] - 1
    ncls = Ww.shape[0]
    h = Wa.shape[0]
    d1 = W1.shape[0]
    d2 = W2.shape[0]
    blk = 1024
    nsteps = n_tok // blk
    dbig = 2 * h + d1

    body = functools.partial(_fused_body, blk=blk, nsteps=nsteps, nseg=nseg,
                             ncls=ncls, h=h, d1=d1)
    row = lambda v: v.reshape(1, -1)
    WbigT = jnp.concatenate([Wa.T, Wg.T, W1.T], axis=1)
    bbig = jnp.concatenate([ba, bg, b1])
    start = cu_seqlens[:nseg].reshape(1, nseg)
    end = cu_seqlens[1:].reshape(1, nseg)
    const = lambda shape: pl.BlockSpec(shape, lambda i: (0, 0))
    out = pl.pallas_call(
        body,
        grid=(nsteps,),
        in_specs=[
            pl.BlockSpec((blk, d), lambda i: (i, 0)),
            const((d, dbig)), const((1, dbig)),
            const((d, ncls)), const((1, ncls)),
            const((d1, d2)), const((1, d2)),
            const((d2, ncls)), const((1, ncls)),
            const((1, nseg)), const((1, nseg)),
        ],
        out_specs=pl.BlockSpec((ncls, nseg), lambda i: (0, 0)),
        out_shape=jax.ShapeDtypeStruct((ncls, nseg), jnp.float32),
        scratch_shapes=[pltpu.VMEM((2 * ncls, nseg), jnp.float32)],
    )(media, WbigT, row(bbig), Ww.T, row(bw),
      W2.T, row(b2), W3.T, row(b3), start, end)
    return out.T * output_scale + output_bias
